# Initial kernel scaffold; baseline (speedup 1.0000x reference)
import jax
import jax.numpy as jnp
from jax.experimental import pallas as pl


def kernel(xe, xe_src, xe_dst, W, M1, M2):
    raise NotImplementedError("write your pallas kernel here")


# speedup vs baseline: 1.2757x; 1.2757x over previous
import functools

import jax
import jax.numpy as jnp
from jax import lax
from jax.experimental import pallas as pl
from jax.experimental.pallas import tpu as pltpu
from jax.experimental.pallas import tpu_sc as plsc

_N = 100000
_NOUT = 100096
_E = 1600000
_EP = 1638400
_V = 32

_EB = 6400
_GRID = _E // _EB


def _premix_body(xe_ref, w_ref, ab_ref, p_ref, q_ref):
    m = xe_ref[:, 0, :] * w_ref[...]
    p_ref[...] = jnp.dot(m, ab_ref[0], preferred_element_type=jnp.float32)
    q_ref[...] = jnp.dot(m, ab_ref[1], preferred_element_type=jnp.float32)


def _premix(xe3, w2, ab):
    return pl.pallas_call(
        _premix_body,
        grid=(_GRID,),
        in_specs=[
            pl.BlockSpec((_EB, 1, _V), lambda i: (i, 0, 0)),
            pl.BlockSpec((_EB, _V), lambda i: (i, 0)),
            pl.BlockSpec((2, _V, _V), lambda i: (0, 0, 0)),
        ],
        out_specs=[
            pl.BlockSpec((_EB, _V), lambda i: (i, 0)),
            pl.BlockSpec((_EB, _V), lambda i: (i, 0)),
        ],
        out_shape=[
            jax.ShapeDtypeStruct((_EP, _V), jnp.float32),
            jax.ShapeDtypeStruct((_EP, _V), jnp.float32),
        ],
    )(xe3, w2, ab)


_NC = 2
_NS = 16
_HALF = _N // _NC
_DUMMY = _HALF
_ACC_ROWS = 50016
_ZROWS = _ACC_ROWS // _NS
_EPT = _EP // _NS
_CH = 128
_JP = _EPT // _CH
_IDXROWS = _EP // _CH
_DROWS = _HALF - (_NS - 1) * _ZROWS

_mesh = plsc.VectorSubcoreMesh(core_axis_name="c", subcore_axis_name="s")


@functools.partial(
    pl.kernel,
    out_type=jax.ShapeDtypeStruct((_NOUT, _V), jnp.float32),
    mesh=_mesh,
    compiler_params=pltpu.CompilerParams(use_tc_tiling_on_sc=False),
    scratch_types=[
        pltpu.VMEM_SHARED((_ACC_ROWS, _V), jnp.float32),
        [pltpu.VMEM((_CH, _V), jnp.float32) for _ in range(3)],
        [pltpu.VMEM((_CH, _V), jnp.float32) for _ in range(3)],
        [pltpu.VMEM((1, _CH), jnp.int32) for _ in range(3)],
        [pltpu.VMEM((1, _CH), jnp.int32) for _ in range(3)],
        [pltpu.SemaphoreType.DMA for _ in range(3)],
        [pltpu.SemaphoreType.DMA for _ in range(3)],
    ],
)
def _scatter_kernel(p_hbm, q_hbm, dst_hbm, src_hbm, zeros_hbm, out_hbm,
                    acc, vp, vq, ip, iq, lsem, ssem):
    c = lax.axis_index("c")
    s = lax.axis_index("s")
    pltpu.sync_copy(zeros_hbm, acc.at[pl.ds(s * _ZROWS, _ZROWS)])
    plsc.subcore_barrier()
    lo = c * _HALF

    def loads(p, b):
        eb = s * _EPT + p * _CH
        r = s * _JP + p
        return (
            pltpu.make_async_copy(p_hbm.at[pl.ds(eb, _CH)], vp[b], lsem[b]),
            pltpu.make_async_copy(q_hbm.at[pl.ds(eb, _CH)], vq[b], lsem[b]),
            pltpu.make_async_copy(dst_hbm.at[pl.ds(r, 1)], ip[b], lsem[b]),
            pltpu.make_async_copy(src_hbm.at[pl.ds(r, 1)], iq[b], lsem[b]),
        )

    def scatters(b):
        return (
            pltpu.make_async_copy(vp[b], acc.at[ip[b].at[0]], ssem[b]),
            pltpu.make_async_copy(vq[b], acc.at[iq[b].at[0]], ssem[b]),
        )

    for d in loads(0, 0):
        d.start()

    def outer(k, carry):
        for q in range(3):
            p = 3 * k + q

            @pl.when(p < _JP)
            def _step():
                @pl.when(p >= 2)
                def _drain():
                    for d in scatters((q + 1) % 3):
                        d.wait()

                @pl.when(p + 1 < _JP)
                def _prefetch():
                    for d in loads(p + 1, (q + 1) % 3):
                        d.start()

                for d in loads(p, q):
                    d.wait()
                for kk in range(_CH // 16):
                    sl = pl.ds(kk * 16, 16)
                    v = ip[q][0, sl] - lo
                    ip[q][0, sl] = jnp.where((v >= 0) & (v < _HALF), v, _DUMMY)
                    u = iq[q][0, sl] - lo
                    iq[q][0, sl] = jnp.where((u >= 0) & (u < _HALF), u, _DUMMY)
                pltpu.async_copy(vp[q], acc.at[ip[q].at[0]], ssem[q], add=True)
                pltpu.async_copy(vq[q], acc.at[iq[q].at[0]], ssem[q], add=True)

        return carry

    lax.fori_loop(0, (_JP + 2) // 3, outer, 0)
    for b in (0, 1):
        for d in scatters(b):
            d.wait()
    plsc.subcore_barrier()
    obase = c * _HALF + s * _ZROWS

    @pl.when(s < _NS - 1)
    def _dump_full():
        pltpu.sync_copy(acc.at[pl.ds(s * _ZROWS, _ZROWS)],
                        out_hbm.at[pl.ds(obase, _ZROWS)])

    @pl.when(s == _NS - 1)
    def _dump_tail():
        pltpu.sync_copy(acc.at[pl.ds(s * _ZROWS, _DROWS)],
                        out_hbm.at[pl.ds(obase, _DROWS)])


def _pad_idx(idx):
    pad = jnp.full((_EP - _E,), _N, jnp.int32)
    return jnp.concatenate([idx.astype(jnp.int32), pad]).reshape(_IDXROWS, _CH)


def kernel(xe, xe_src, xe_dst, W, M1, M2):
    a = (M1 + M2) * 0.5
    b = (M2 - M1) * 0.5
    ab = jnp.stack([a, b])
    p, q = _premix(xe, W, ab)
    zeros = jnp.zeros((_ZROWS, _V), jnp.float32)
    out = _scatter_kernel(p, q, _pad_idx(xe_dst), _pad_idx(xe_src), zeros)
    return out[:_N].reshape(_N, 1, _V)


# speedup vs baseline: 2.2028x; 1.7268x over previous
import functools

import jax
import jax.numpy as jnp
from jax import lax
from jax.experimental import pallas as pl
from jax.experimental.pallas import tpu as pltpu
from jax.experimental.pallas import tpu_sc as plsc

_N = 100000
_NOUT = 100096
_E = 1600000
_EP = 1638400
_V = 32

_PACK = 4
_E4 = _E // _PACK
_EP4 = _EP // _PACK
_EB4 = 3200
_GRID = _E4 // _EB4


def _premix_body(xe_ref, w_ref, ab_ref, p_ref, q_ref):
    m = xe_ref[...] * w_ref[...]
    p_ref[...] = jnp.dot(m, ab_ref[0], preferred_element_type=jnp.float32)
    q_ref[...] = jnp.dot(m, ab_ref[1], preferred_element_type=jnp.float32)


def _premix(xe4, w4, ab):
    return pl.pallas_call(
        _premix_body,
        grid=(_GRID,),
        in_specs=[
            pl.BlockSpec((_EB4, 128), lambda i: (i, 0)),
            pl.BlockSpec((_EB4, 128), lambda i: (i, 0)),
            pl.BlockSpec((2, 128, 128), lambda i: (0, 0, 0)),
        ],
        out_specs=[
            pl.BlockSpec((_EB4, 128), lambda i: (i, 0)),
            pl.BlockSpec((_EB4, 128), lambda i: (i, 0)),
        ],
        out_shape=[
            jax.ShapeDtypeStruct((_EP4, 128), jnp.float32),
            jax.ShapeDtypeStruct((_EP4, 128), jnp.float32),
        ],
    )(xe4, w4, ab)


_NC = 2
_NS = 16
_HALF = _N // _NC
_DUMMY = _HALF
_ACC_ROWS = 50016
_ZROWS = _ACC_ROWS // _NS
_EPT = _EP // _NS
_CH = 128
_JP = _EPT // _CH
_IDXROWS = _EP // _CH
_DROWS = _HALF - (_NS - 1) * _ZROWS

_mesh = plsc.VectorSubcoreMesh(core_axis_name="c", subcore_axis_name="s")


@functools.partial(
    pl.kernel,
    out_type=jax.ShapeDtypeStruct((_NOUT, _V), jnp.float32),
    mesh=_mesh,
    compiler_params=pltpu.CompilerParams(use_tc_tiling_on_sc=False),
    scratch_types=[
        pltpu.VMEM_SHARED((_ACC_ROWS, _V), jnp.float32),
        [pltpu.VMEM((_CH, _V), jnp.float32) for _ in range(3)],
        [pltpu.VMEM((_CH, _V), jnp.float32) for _ in range(3)],
        [pltpu.VMEM((1, _CH), jnp.int32) for _ in range(3)],
        [pltpu.VMEM((1, _CH), jnp.int32) for _ in range(3)],
        [pltpu.SemaphoreType.DMA for _ in range(3)],
        [pltpu.SemaphoreType.DMA for _ in range(3)],
    ],
)
def _scatter_kernel(p_hbm, q_hbm, dst_hbm, src_hbm, zeros_hbm, out_hbm,
                    acc, vp, vq, ip, iq, lsem, ssem):
    c = lax.axis_index("c")
    s = lax.axis_index("s")
    pltpu.sync_copy(zeros_hbm, acc.at[pl.ds(s * _ZROWS, _ZROWS)])
    plsc.subcore_barrier()
    lo = c * _HALF

    def loads(p, b):
        eb = s * _EPT + p * _CH
        r = s * _JP + p
        return (
            pltpu.make_async_copy(p_hbm.at[pl.ds(eb, _CH)], vp[b], lsem[b]),
            pltpu.make_async_copy(q_hbm.at[pl.ds(eb, _CH)], vq[b], lsem[b]),
            pltpu.make_async_copy(dst_hbm.at[pl.ds(r, 1)], ip[b], lsem[b]),
            pltpu.make_async_copy(src_hbm.at[pl.ds(r, 1)], iq[b], lsem[b]),
        )

    def scatters(b):
        return (
            pltpu.make_async_copy(vp[b], acc.at[ip[b].at[0]], ssem[b]),
            pltpu.make_async_copy(vq[b], acc.at[iq[b].at[0]], ssem[b]),
        )

    for d in loads(0, 0):
        d.start()

    def outer(k, carry):
        for q in range(3):
            p = 3 * k + q

            @pl.when(p < _JP)
            def _step():
                @pl.when(p >= 2)
                def _drain():
                    for d in scatters((q + 1) % 3):
                        d.wait()

                @pl.when(p + 1 < _JP)
                def _prefetch():
                    for d in loads(p + 1, (q + 1) % 3):
                        d.start()

                for d in loads(p, q):
                    d.wait()
                for kk in range(_CH // 16):
                    sl = pl.ds(kk * 16, 16)
                    v = ip[q][0, sl] - lo
                    ip[q][0, sl] = jnp.where((v >= 0) & (v < _HALF), v, _DUMMY)
                    u = iq[q][0, sl] - lo
                    iq[q][0, sl] = jnp.where((u >= 0) & (u < _HALF), u, _DUMMY)
                pltpu.async_copy(vp[q], acc.at[ip[q].at[0]], ssem[q], add=True)
                pltpu.async_copy(vq[q], acc.at[iq[q].at[0]], ssem[q], add=True)

        return carry

    lax.fori_loop(0, (_JP + 2) // 3, outer, 0)
    for b in (0, 1):
        for d in scatters(b):
            d.wait()
    plsc.subcore_barrier()
    obase = c * _HALF + s * _ZROWS

    @pl.when(s < _NS - 1)
    def _dump_full():
        pltpu.sync_copy(acc.at[pl.ds(s * _ZROWS, _ZROWS)],
                        out_hbm.at[pl.ds(obase, _ZROWS)])

    @pl.when(s == _NS - 1)
    def _dump_tail():
        pltpu.sync_copy(acc.at[pl.ds(s * _ZROWS, _DROWS)],
                        out_hbm.at[pl.ds(obase, _DROWS)])


def _pad_idx(idx):
    pad = jnp.full((_EP - _E,), _N, jnp.int32)
    return jnp.concatenate([idx.astype(jnp.int32), pad]).reshape(_IDXROWS, _CH)


def kernel(xe, xe_src, xe_dst, W, M1, M2):
    a = (M1 + M2) * 0.5
    b = (M2 - M1) * 0.5
    eye4 = jnp.eye(_PACK, dtype=jnp.float32)
    ab = jnp.stack([jnp.kron(eye4, a), jnp.kron(eye4, b)])
    p4, q4 = _premix(xe.reshape(_E4, 128), W.reshape(_E4, 128), ab)
    p = p4.reshape(_EP, _V)
    q = q4.reshape(_EP, _V)
    zeros = jnp.zeros((_ZROWS, _V), jnp.float32)
    out = _scatter_kernel(p, q, _pad_idx(xe_dst), _pad_idx(xe_src), zeros)
    return out[:_N].reshape(_N, 1, _V)
